# baseline (speedup 1.0000x reference)
import jax
import jax.numpy as jnp
from jax import lax
from jax.experimental import pallas as pl
from jax.experimental.pallas import tpu as pltpu
from jax.experimental.pallas import tpu_sc as plsc

VOCAB = 1000000
EMB = 32
N_NEGS = 5
BATCH = 16384
CTX = 20
PAIRS = CTX * (1 + N_NEGS)
TOTAL = BATCH * PAIRS

NC = 2
NS = 16
NW = NC * NS
B_PER_W = BATCH // NW
CB = 16
CHUNKS = B_PER_W // CB
CP = CB * PAIRS
IDX_ROWS = CP // 128
W_ROWS = CHUNKS * IDX_ROWS
RING = 384
PACK = VOCAB // 4


def _sc_scores_body(w2, wm, iw2, iwm, ivp, ovp, scores_hbm,
                    w2_buf, wm_buf, iw2_buf, iwm_buf, iv_pad, ov_ring,
                    sc_buf, sem_iv, g_sem0, g_sem1, g_sem2):
    g_sems = [g_sem0, g_sem1, g_sem2]
    wid = lax.axis_index("s") * NC + lax.axis_index("c")
    lane1 = lax.iota(jnp.int32, 16)

    pltpu.sync_copy(iw2.at[wid], iw2_buf)
    pltpu.sync_copy(iwm.at[wid], iwm_buf)
    ivcps = [pltpu.async_copy(ivp.at[iw2_buf.at[r]],
                              iv_pad.at[pl.ds(r * 128, 128)], sem_iv)
             for r in range(4)]
    for cp in ivcps:
        cp.wait()

    def compute_b(c, bi):
        bl = c * CB + bi
        blv = jnp.full((16,), bl, jnp.int32)
        isubv = plsc.load_gather(
            iwm_buf, [blv >> 7, blv & 127]) * 32
        rbase = (bi * PAIRS) % RING

        def g_body(g, carry):
            gs = jnp.minimum(g * 16, PAIRS - 16)
            pb = bi * PAIRS + gs
            pbv = jnp.full((16,), pb, jnp.int32) + lane1
            posv = jnp.full((16,), rbase, jnp.int32) + gs + lane1
            posv = jnp.where(posv >= RING, posv - RING, posv)
            subv = plsc.load_gather(wm_buf, [pbv >> 7, pbv & 127]) * 32
            acc = jnp.zeros((16,), jnp.float32)
            for d in range(32):
                ov_d = plsc.load_gather(ov_ring, [posv, subv + d])
                iv_d = plsc.load_gather(iv_pad, [blv, isubv + d])
                acc = acc + ov_d * iv_d
            sc_buf[pl.ds(pb, 16)] = acc
            return carry

        lax.fori_loop(0, 8, g_body, 0)

    pltpu.sync_copy(w2.at[wid * CHUNKS], w2_buf)
    for j in range(2):
        pltpu.async_copy(ovp.at[w2_buf.at[j]],
                         ov_ring.at[pl.ds(j * 128, 128)], g_sems[j])

    def chunk_body(c, carry):
        pltpu.sync_copy(w2.at[wid * CHUNKS + c], w2_buf)
        pltpu.sync_copy(wm.at[wid * CHUNKS + c], wm_buf)
        for j in range(IDX_ROWS):
            pltpu.make_async_copy(
                ovp.at[w2_buf.at[0]],
                ov_ring.at[pl.ds((j % 3) * 128, 128)],
                g_sems[j % 3]).wait()
            compute_b(c, j)
            if j == IDX_ROWS - 1:
                compute_b(c, CB - 1)
            r = c * IDX_ROWS + j + 2

            @pl.when(r < W_ROWS)
            def _():
                pltpu.async_copy(
                    ovp.at[w2_buf.at[j + 2]],
                    ov_ring.at[pl.ds(((j + 2) % 3) * 128, 128)],
                    g_sems[(j + 2) % 3])
        pltpu.sync_copy(sc_buf,
                        scores_hbm.at[pl.ds(wid * (CHUNKS * CP) + c * CP, CP)])
        return carry

    lax.fori_loop(0, CHUNKS, chunk_body, 0)


_sc_scores = pl.kernel(
    _sc_scores_body,
    out_type=jax.ShapeDtypeStruct((TOTAL,), jnp.float32),
    mesh=plsc.VectorSubcoreMesh(core_axis_name="c", subcore_axis_name="s"),
    compiler_params=pltpu.CompilerParams(needs_layout_passes=False,
                                         use_tc_tiling_on_sc=True),
    scratch_types=[
        pltpu.VMEM((24, 128), jnp.int32),
        pltpu.VMEM((24, 128), jnp.int32),
        pltpu.VMEM((8, 128), jnp.int32),
        pltpu.VMEM((8, 128), jnp.int32),
        pltpu.VMEM((B_PER_W, 128), jnp.float32),
        pltpu.VMEM((RING, 128), jnp.float32),
        pltpu.VMEM((CP,), jnp.float32),
        pltpu.SemaphoreType.DMA,
        pltpu.SemaphoreType.DMA,
        pltpu.SemaphoreType.DMA,
        pltpu.SemaphoreType.DMA,
    ],
)


def _tc_loss_body(s_ref, o_ref):
    s = s_ref[...]
    rows = lax.broadcasted_iota(jnp.int32, s.shape, 0)
    cols = lax.broadcasted_iota(jnp.int32, s.shape, 1)
    k = (rows * 128 + cols) % PAIRS
    z = jnp.where(k < CTX, s, -s)
    t = jax.nn.log_sigmoid(z)
    o_ref[0, 0] = -jnp.sum(t) / (BATCH * CTX)


def _tc_loss(s2d):
    return pl.pallas_call(
        _tc_loss_body,
        out_shape=jax.ShapeDtypeStruct((1, 1), jnp.float32),
        out_specs=pl.BlockSpec(memory_space=pltpu.SMEM),
    )(s2d)


def kernel(iword, owords, ivec_w, ovec_w):
    nwords = jax.random.randint(jax.random.key(12345),
                                (BATCH, CTX * N_NEGS), 0, VOCAB)
    widx = jnp.concatenate(
        [owords.astype(jnp.int32), nwords.astype(jnp.int32)], axis=1)
    w4 = widx.reshape(NW, CHUNKS, IDX_ROWS, 128)
    extra = jnp.roll(w4, -1, axis=1)[:, :, :2, :]
    w17 = jnp.concatenate([w4, extra], axis=2)
    w24 = jnp.pad(w17, ((0, 0), (0, 0), (0, 7), (0, 0)))
    w24 = w24.reshape(NW * CHUNKS, 24, 128)
    w2 = w24 >> 2
    wm = w24 & 3
    iw = iword.astype(jnp.int32).reshape(NW, 4, 128)
    iw8 = jnp.pad(iw, ((0, 0), (0, 4), (0, 0)))
    iw2 = iw8 >> 2
    iwm = iw8 & 3
    ivp = ivec_w.reshape(PACK, 128)
    ovp = ovec_w.reshape(PACK, 128)
    scores = _sc_scores(w2, wm, iw2, iwm, ivp, ovp)
    loss = _tc_loss(scores.reshape(TOTAL // 128, 128))
    return loss[0, 0]


# speedup vs baseline: 1.4926x; 1.4926x over previous
import jax
import jax.numpy as jnp
from jax import lax
from jax.experimental import pallas as pl
from jax.experimental.pallas import tpu as pltpu
from jax.experimental.pallas import tpu_sc as plsc

VOCAB = 1000000
EMB = 32
N_NEGS = 5
BATCH = 16384
CTX = 20
PAIRS = CTX * (1 + N_NEGS)
TOTAL = BATCH * PAIRS

NC = 2
NS = 16
NW = NC * NS
B_PER_W = BATCH // NW
CB = 16
CHUNKS = B_PER_W // CB
CP = CB * PAIRS
IDX_ROWS = CP // 128
W_ROWS = CHUNKS * IDX_ROWS
RING = 640


def _sc_scores_body(widx3d, iw3d, ivtbl, ovtbl, scores_hbm,
                    widx_buf, iw_buf, iv_buf, ov_ring, part, sc_buf,
                    sem_iv, g_sem0, g_sem1, g_sem2, g_sem3, g_sem4):
    g_sems = [g_sem0, g_sem1, g_sem2, g_sem3, g_sem4]
    wid = lax.axis_index("s") * NC + lax.axis_index("c")
    lane16 = lax.iota(jnp.int32, 16) * 16

    pltpu.sync_copy(iw3d.at[wid], iw_buf)
    ivcps = [pltpu.async_copy(ivtbl.at[iw_buf.at[r]],
                              iv_buf.at[pl.ds(r * 128, 128)], sem_iv)
             for r in range(4)]
    for cp in ivcps:
        cp.wait()
    pltpu.sync_copy(widx3d.at[wid], widx_buf)

    def issue(r, slot):
        return pltpu.async_copy(
            ovtbl.at[widx_buf.at[r]],
            ov_ring.at[pl.ds(slot * 128, 128)],
            g_sems[slot])

    def compute_b(c, bi):
        iv0 = iv_buf[c * CB + bi, pl.ds(0, 16)]
        iv1 = iv_buf[c * CB + bi, pl.ds(16, 16)]
        rbase = (bi * PAIRS) % RING

        def g_body(g, carry):
            gs = jnp.minimum(g * 16, PAIRS - 16)
            gpos = rbase + gs
            gpos = jnp.where(gpos >= RING, gpos - RING, gpos)
            for jj in range(16):
                pos = gpos + jj
                pos = jnp.where(pos >= RING, pos - RING, pos)
                h0 = ov_ring[pos, pl.ds(0, 16)]
                h1 = ov_ring[pos, pl.ds(16, 16)]
                part[pl.ds(jj * 16, 16)] = h0 * iv0 + h1 * iv1
            acc = jnp.zeros((16,), jnp.float32)
            for d in range(16):
                acc = acc + plsc.load_gather(part, [lane16 + d])
            sc_buf[pl.ds(bi * PAIRS + gs, 16)] = acc
            return carry

        lax.fori_loop(0, 8, g_body, 0)

    for j in range(4):
        issue(j, j)

    def chunk_body(c, carry):
        for j in range(IDX_ROWS):
            pltpu.make_async_copy(
                ovtbl.at[widx_buf.at[0]],
                ov_ring.at[pl.ds((j % 5) * 128, 128)],
                g_sems[j % 5]).wait()
            compute_b(c, j)
            if j == IDX_ROWS - 1:
                compute_b(c, CB - 1)
            r = c * IDX_ROWS + j + 4

            @pl.when(r < W_ROWS)
            def _():
                issue(r, (j + 4) % 5)
        pltpu.sync_copy(sc_buf,
                        scores_hbm.at[pl.ds(wid * (CHUNKS * CP) + c * CP, CP)])
        return carry

    lax.fori_loop(0, CHUNKS, chunk_body, 0)


_sc_scores = pl.kernel(
    _sc_scores_body,
    out_type=jax.ShapeDtypeStruct((TOTAL,), jnp.float32),
    mesh=plsc.VectorSubcoreMesh(core_axis_name="c", subcore_axis_name="s"),
    compiler_params=pltpu.CompilerParams(needs_layout_passes=False,
                                         use_tc_tiling_on_sc=False),
    scratch_types=[
        pltpu.VMEM((W_ROWS, 128), jnp.int32),
        pltpu.VMEM((4, 128), jnp.int32),
        pltpu.VMEM((B_PER_W, EMB), jnp.float32),
        pltpu.VMEM((RING, EMB), jnp.float32),
        pltpu.VMEM((256,), jnp.float32),
        pltpu.VMEM((CP,), jnp.float32),
        pltpu.SemaphoreType.DMA,
        pltpu.SemaphoreType.DMA,
        pltpu.SemaphoreType.DMA,
        pltpu.SemaphoreType.DMA,
        pltpu.SemaphoreType.DMA,
        pltpu.SemaphoreType.DMA,
    ],
)


def _tc_loss_body(s_ref, o_ref):
    s = s_ref[...]
    rows = lax.broadcasted_iota(jnp.int32, s.shape, 0)
    cols = lax.broadcasted_iota(jnp.int32, s.shape, 1)
    k = (rows * 128 + cols) % PAIRS
    z = jnp.where(k < CTX, s, -s)
    t = jax.nn.log_sigmoid(z)
    o_ref[0, 0] = -jnp.sum(t) / (BATCH * CTX)


def _tc_loss(s2d):
    return pl.pallas_call(
        _tc_loss_body,
        out_shape=jax.ShapeDtypeStruct((1, 1), jnp.float32),
        out_specs=pl.BlockSpec(memory_space=pltpu.SMEM),
    )(s2d)


def kernel(iword, owords, ivec_w, ovec_w):
    nwords = jax.random.randint(jax.random.key(12345),
                                (BATCH, CTX * N_NEGS), 0, VOCAB)
    widx = jnp.concatenate(
        [owords.astype(jnp.int32), nwords.astype(jnp.int32)], axis=1)
    widx3d = widx.reshape(NW, W_ROWS, 128)
    iw3d = iword.astype(jnp.int32).reshape(NW, 4, 128)
    scores = _sc_scores(widx3d, iw3d, ivec_w, ovec_w)
    loss = _tc_loss(scores.reshape(TOTAL // 128, 128))
    return loss[0, 0]
